# baseline (speedup 1.0000x reference)
import functools

import jax
import jax.numpy as jnp
from jax import lax
from jax.experimental import pallas as pl
from jax.experimental.pallas import tpu as pltpu
from jax.experimental.pallas import tpu_sc as plsc

N = 10000
E = 320000
D = 128
C = 10

NC = 2
NS = 16
NW = NC * NS
K = 128
NCHUNK = 80
EPW = K * NCHUNK
EPAD = NW * EPW
NP = 10240
RPT = NP // NS
RCH = 128
DPT = NP // NS

_mesh = plsc.VectorSubcoreMesh(
    core_axis_name="c", subcore_axis_name="s", num_cores=NC, num_subcores=NS)


@functools.partial(
    pl.kernel,
    mesh=_mesh,
    out_type=jax.ShapeDtypeStruct((NC, 2, NP), jnp.float32),
    scratch_types=[
        pltpu.VMEM((2, K), jnp.int32),
        pltpu.VMEM((K,), jnp.float32),
        pltpu.VMEM((DPT,), jnp.float32),
        pltpu.VMEM_SHARED((NP,), jnp.float32),
        pltpu.VMEM_SHARED((NP,), jnp.float32),
    ],
)
def _degrees(sd_hbm, out_hbm, sdb, ones_v, buf, odeg, ideg):
    cid = lax.axis_index("c")
    sid = lax.axis_index("s")
    wid = cid * NS + sid

    def zb(i, _):
        buf[pl.ds(i * 16, 16)] = jnp.zeros((16,), jnp.float32)
        return 0
    lax.fori_loop(0, DPT // 16, zb, 0)

    def ob(i, _):
        ones_v[pl.ds(i * 16, 16)] = jnp.ones((16,), jnp.float32)
        return 0
    lax.fori_loop(0, K // 16, ob, 0)

    base = sid * DPT
    pltpu.sync_copy(buf, odeg.at[pl.ds(base, DPT)])
    pltpu.sync_copy(buf, ideg.at[pl.ds(base, DPT)])
    plsc.subcore_barrier()

    def step(c, _):
        pltpu.sync_copy(sd_hbm.at[wid, c], sdb)
        pltpu.sync_copy(ones_v, odeg.at[sdb.at[0]], add=True)
        pltpu.sync_copy(ones_v, ideg.at[sdb.at[1]], add=True)
        return 0
    lax.fori_loop(0, NCHUNK, step, 0)
    plsc.subcore_barrier()

    pltpu.sync_copy(odeg.at[pl.ds(base, DPT)], buf)
    pltpu.sync_copy(buf, out_hbm.at[cid, 0, pl.ds(base, DPT)])
    pltpu.sync_copy(ideg.at[pl.ds(base, DPT)], buf)
    pltpu.sync_copy(buf, out_hbm.at[cid, 1, pl.ds(base, DPT)])


@functools.partial(
    pl.kernel,
    mesh=_mesh,
    out_type=jax.ShapeDtypeStruct((NC, NP, D), jnp.float32),
    scratch_types=[
        pltpu.VMEM((2, K), jnp.int32),
        pltpu.VMEM((2, K), jnp.int32),
        pltpu.VMEM((K, D), jnp.float32),
        pltpu.VMEM((K, D), jnp.float32),
        pltpu.VMEM_SHARED((NP, D), jnp.float32),
        pltpu.SemaphoreType.DMA,
        pltpu.SemaphoreType.DMA,
    ],
)
def _propagate(h_hbm, sd_hbm, out_hbm, idxA, idxB, rows0, rows1, acc, sem0, sem1):
    cid = lax.axis_index("c")
    sid = lax.axis_index("s")
    wid = cid * NS + sid

    def zb(i, _):
        rows0[i // 8, pl.ds((i % 8) * 16, 16)] = jnp.zeros((16,), jnp.float32)
        return 0
    lax.fori_loop(0, RCH * (D // 16), zb, 0)

    base = sid * RPT
    for r in range(RPT // RCH):
        pltpu.sync_copy(rows0, acc.at[pl.ds(base + r * RCH, RCH)])
    plsc.subcore_barrier()

    pltpu.sync_copy(sd_hbm.at[wid, 0], idxA)
    pltpu.async_copy(h_hbm.at[idxA.at[0]], rows0, sem0)

    def body(j, _):
        b = 2 * j + 1
        pltpu.sync_copy(sd_hbm.at[wid, b], idxB)
        pltpu.async_copy(h_hbm.at[idxB.at[0]], rows1, sem1)
        pltpu.make_async_copy(h_hbm.at[idxA.at[0]], rows0, sem0).wait()
        pltpu.sync_copy(rows0, acc.at[idxA.at[1]], add=True)

        @pl.when(j < NCHUNK // 2 - 1)
        def _():
            pltpu.sync_copy(sd_hbm.at[wid, b + 1], idxA)
            pltpu.async_copy(h_hbm.at[idxA.at[0]], rows0, sem0)

        pltpu.make_async_copy(h_hbm.at[idxB.at[0]], rows1, sem1).wait()
        pltpu.sync_copy(rows1, acc.at[idxB.at[1]], add=True)
        return 0
    lax.fori_loop(0, NCHUNK // 2, body, 0)
    plsc.subcore_barrier()

    for r in range(RPT // RCH):
        sl = pl.ds(base + r * RCH, RCH)
        pltpu.sync_copy(acc.at[sl], rows0)
        pltpu.sync_copy(rows0, out_hbm.at[cid, sl])


def _prep_body(degT_ref, x_ref, h0_ref, ns_ref, nd_ref):
    d = degT_ref[...]
    od = d[:, 0:1] + d[:, 1:2]
    idg = d[:, 2:3] + d[:, 3:4]
    ns = lax.rsqrt(jnp.maximum(od, 1.0))
    nd = lax.rsqrt(jnp.maximum(idg, 1.0))
    h0_ref[...] = x_ref[...] * ns
    ns_ref[...] = ns
    nd_ref[...] = nd


_prep = pl.pallas_call(
    _prep_body,
    out_shape=[
        jax.ShapeDtypeStruct((NP, D), jnp.float32),
        jax.ShapeDtypeStruct((NP, 1), jnp.float32),
        jax.ShapeDtypeStruct((NP, 1), jnp.float32),
    ],
)


def _mid_body(p_ref, ns_ref, nd_ref, w_ref, b_ref, out_ref):
    agg = (p_ref[0] + p_ref[1]) * nd_ref[...]
    z = jnp.dot(agg, w_ref[...], preferred_element_type=jnp.float32) + b_ref[...]
    out_ref[...] = jnp.maximum(z, 0.0) * ns_ref[...]


_mid = pl.pallas_call(
    _mid_body,
    out_shape=jax.ShapeDtypeStruct((NP, D), jnp.float32),
)


def _final_body(p_ref, nd_ref, w_ref, b_ref, wl_ref, bl_ref, out_ref):
    agg = (p_ref[0] + p_ref[1]) * nd_ref[...]
    z = jnp.dot(agg, w_ref[...], preferred_element_type=jnp.float32) + b_ref[...]
    h = jnp.maximum(z, 0.0)
    m = jnp.sum(h[:N], axis=0, keepdims=True) * (1.0 / N)
    lg = jnp.dot(m, wl_ref[...], preferred_element_type=jnp.float32) + bl_ref[...]
    e = jnp.exp(lg - jnp.max(lg, axis=1, keepdims=True))
    out_ref[...] = e / jnp.sum(e, axis=1, keepdims=True)


_final = pl.pallas_call(
    _final_body,
    out_shape=jax.ShapeDtypeStruct((1, C), jnp.float32),
)


def kernel(x, edge_index, W1, b1, W2, b2, Wl, bl):
    pad = jnp.full((EPAD - E,), N, jnp.int32)
    srcp = jnp.concatenate([edge_index[0], pad]).reshape(NW, NCHUNK, K)
    dstp = jnp.concatenate([edge_index[1], pad]).reshape(NW, NCHUNK, K)
    sd = jnp.stack([srcp, dstp], axis=2)
    x_pad = jnp.pad(x, ((0, NP - N), (0, 0)))

    deg_parts = _degrees(sd)
    degT = deg_parts.transpose(2, 1, 0).reshape(NP, 4)

    h0, ns, nd = _prep(degT, x_pad)
    parts1 = _propagate(h0, sd)
    h1 = _mid(parts1, ns, nd, W1, b1.reshape(1, D))
    parts2 = _propagate(h1, sd)
    out = _final(parts2, nd, W2, b2.reshape(1, D), Wl, bl.reshape(1, C))
    return out


# speedup vs baseline: 2.7923x; 2.7923x over previous
import functools

import jax
import jax.numpy as jnp
from jax import lax
from jax.experimental import pallas as pl
from jax.experimental.pallas import tpu as pltpu
from jax.experimental.pallas import tpu_sc as plsc

N = 10000
E = 320000
D = 128
C = 10

NC = 2
NS = 16
NW = NC * NS
K = 64
NCHUNK = 160
NSLOT = 4
NITER = NCHUNK // NSLOT
EPW = K * NCHUNK
EPAD = NW * EPW
NP = 10240
RPT = NP // NS
RCH = K
DPT = NP // NS

_mesh = plsc.VectorSubcoreMesh(
    core_axis_name="c", subcore_axis_name="s", num_cores=NC, num_subcores=NS)


@functools.partial(
    pl.kernel,
    mesh=_mesh,
    out_type=jax.ShapeDtypeStruct((NC, 2, NP), jnp.float32),
    scratch_types=[
        pltpu.VMEM((NCHUNK, 2, K), jnp.int32),
        pltpu.VMEM((K,), jnp.float32),
        pltpu.VMEM((DPT,), jnp.float32),
        pltpu.VMEM_SHARED((NP,), jnp.float32),
        pltpu.VMEM_SHARED((NP,), jnp.float32),
        pltpu.SemaphoreType.DMA,
        pltpu.SemaphoreType.DMA,
    ],
)
def _degrees(sd_hbm, out_hbm, idxall, ones_v, buf, odeg, ideg, sem_o, sem_i):
    cid = lax.axis_index("c")
    sid = lax.axis_index("s")
    wid = cid * NS + sid

    def zb(i, _):
        buf[pl.ds(i * 16, 16)] = jnp.zeros((16,), jnp.float32)
        return 0
    lax.fori_loop(0, DPT // 16, zb, 0)

    def ob(i, _):
        ones_v[pl.ds(i * 16, 16)] = jnp.ones((16,), jnp.float32)
        return 0
    lax.fori_loop(0, K // 16, ob, 0)

    base = sid * DPT
    pltpu.sync_copy(buf, odeg.at[pl.ds(base, DPT)])
    pltpu.sync_copy(buf, ideg.at[pl.ds(base, DPT)])
    plsc.subcore_barrier()

    pltpu.sync_copy(sd_hbm.at[wid], idxall)

    GRP = 8

    def step(g, _):
        for t in range(GRP):
            c = g * GRP + t
            pltpu.async_copy(ones_v, odeg.at[idxall.at[c, 0]], sem_o, add=True)
            pltpu.async_copy(ones_v, ideg.at[idxall.at[c, 1]], sem_i, add=True)
        for t in range(GRP):
            c = g * GRP + t
            pltpu.make_async_copy(ones_v, odeg.at[idxall.at[c, 0]], sem_o).wait()
            pltpu.make_async_copy(ones_v, ideg.at[idxall.at[c, 1]], sem_i).wait()
        return 0
    lax.fori_loop(0, NCHUNK // GRP, step, 0)
    plsc.subcore_barrier()

    pltpu.sync_copy(odeg.at[pl.ds(base, DPT)], buf)
    pltpu.sync_copy(buf, out_hbm.at[cid, 0, pl.ds(base, DPT)])
    pltpu.sync_copy(ideg.at[pl.ds(base, DPT)], buf)
    pltpu.sync_copy(buf, out_hbm.at[cid, 1, pl.ds(base, DPT)])


@functools.partial(
    pl.kernel,
    mesh=_mesh,
    out_type=jax.ShapeDtypeStruct((NC, NP, D), jnp.float32),
    scratch_types=[
        [pltpu.VMEM((2, K), jnp.int32) for _ in range(NSLOT)],
        [pltpu.VMEM((K, D), jnp.float32) for _ in range(NSLOT)],
        [pltpu.SemaphoreType.DMA for _ in range(NSLOT)],
        [pltpu.SemaphoreType.DMA for _ in range(NSLOT)],
        [pltpu.SemaphoreType.DMA for _ in range(NSLOT)],
        pltpu.VMEM_SHARED((NP, D), jnp.float32),
    ],
)
def _propagate(h_hbm, sd_hbm, out_hbm, idx, rows, isem, gsem, ssem, acc):
    cid = lax.axis_index("c")
    sid = lax.axis_index("s")
    wid = cid * NS + sid

    def zb(i, _):
        rows[0][i // 8, pl.ds((i % 8) * 16, 16)] = jnp.zeros((16,), jnp.float32)
        return 0
    lax.fori_loop(0, RCH * (D // 16), zb, 0)

    base = sid * RPT
    for r in range(RPT // RCH):
        pltpu.sync_copy(rows[0], acc.at[pl.ds(base + r * RCH, RCH)])
    plsc.subcore_barrier()

    for s in range(NSLOT):
        pltpu.async_copy(sd_hbm.at[wid, s], idx[s], isem[s])
    for s in range(NSLOT):
        pltpu.make_async_copy(sd_hbm.at[wid, s], idx[s], isem[s]).wait()
        pltpu.async_copy(h_hbm.at[idx[s].at[0]], rows[s], gsem[s])

    def body(j, _):
        for s in range(NSLOT):
            pltpu.make_async_copy(h_hbm.at[idx[s].at[0]], rows[s], gsem[s]).wait()
            pltpu.async_copy(rows[s], acc.at[idx[s].at[1]], ssem[s], add=True)

        @pl.when(j < NITER - 1)
        def _():
            c0 = (j + 1) * NSLOT
            for s in range(NSLOT):
                pltpu.make_async_copy(rows[s], acc.at[idx[s].at[1]], ssem[s]).wait()
                pltpu.async_copy(sd_hbm.at[wid, c0 + s], idx[s], isem[s])
            for s in range(NSLOT):
                pltpu.make_async_copy(sd_hbm.at[wid, c0 + s], idx[s], isem[s]).wait()
                pltpu.async_copy(h_hbm.at[idx[s].at[0]], rows[s], gsem[s])
        return 0
    lax.fori_loop(0, NITER, body, 0)
    for s in range(NSLOT):
        pltpu.make_async_copy(rows[s], acc.at[idx[s].at[1]], ssem[s]).wait()
    plsc.subcore_barrier()

    for r in range(RPT // RCH):
        sl = pl.ds(base + r * RCH, RCH)
        pltpu.sync_copy(acc.at[sl], rows[0])
        pltpu.sync_copy(rows[0], out_hbm.at[cid, sl])


def _prep_body(degT_ref, x_ref, h0_ref, ns_ref, nd_ref):
    d = degT_ref[...]
    od = d[:, 0:1] + d[:, 1:2]
    idg = d[:, 2:3] + d[:, 3:4]
    ns = lax.rsqrt(jnp.maximum(od, 1.0))
    nd = lax.rsqrt(jnp.maximum(idg, 1.0))
    h0_ref[...] = x_ref[...] * ns
    ns_ref[...] = ns
    nd_ref[...] = nd


_prep = pl.pallas_call(
    _prep_body,
    out_shape=[
        jax.ShapeDtypeStruct((NP, D), jnp.float32),
        jax.ShapeDtypeStruct((NP, 1), jnp.float32),
        jax.ShapeDtypeStruct((NP, 1), jnp.float32),
    ],
)


def _mid_body(p_ref, ns_ref, nd_ref, w_ref, b_ref, out_ref):
    agg = (p_ref[0] + p_ref[1]) * nd_ref[...]
    z = jnp.dot(agg, w_ref[...], preferred_element_type=jnp.float32) + b_ref[...]
    out_ref[...] = jnp.maximum(z, 0.0) * ns_ref[...]


_mid = pl.pallas_call(
    _mid_body,
    out_shape=jax.ShapeDtypeStruct((NP, D), jnp.float32),
)


def _final_body(p_ref, nd_ref, w_ref, b_ref, wl_ref, bl_ref, out_ref):
    agg = (p_ref[0] + p_ref[1]) * nd_ref[...]
    z = jnp.dot(agg, w_ref[...], preferred_element_type=jnp.float32) + b_ref[...]
    h = jnp.maximum(z, 0.0)
    m = jnp.sum(h[:N], axis=0, keepdims=True) * (1.0 / N)
    lg = jnp.dot(m, wl_ref[...], preferred_element_type=jnp.float32) + bl_ref[...]
    e = jnp.exp(lg - jnp.max(lg, axis=1, keepdims=True))
    out_ref[...] = e / jnp.sum(e, axis=1, keepdims=True)


_final = pl.pallas_call(
    _final_body,
    out_shape=jax.ShapeDtypeStruct((1, C), jnp.float32),
)


def kernel(x, edge_index, W1, b1, W2, b2, Wl, bl):
    pad = N + (jnp.arange(EPAD - E, dtype=jnp.int32) % (NP - N))
    srcp = jnp.concatenate([edge_index[0], pad]).reshape(NW, NCHUNK, K)
    dstp = jnp.concatenate([edge_index[1], pad]).reshape(NW, NCHUNK, K)
    sd = jnp.stack([srcp, dstp], axis=2)
    x_pad = jnp.pad(x, ((0, NP - N), (0, 0)))

    deg_parts = _degrees(sd)
    degT = deg_parts.transpose(2, 1, 0).reshape(NP, 4)

    h0, ns, nd = _prep(degT, x_pad)
    parts1 = _propagate(h0, sd)
    h1 = _mid(parts1, ns, nd, W1, b1.reshape(1, D))
    parts2 = _propagate(h1, sd)
    out = _final(parts2, nd, W2, b2.reshape(1, D), Wl, bl.reshape(1, C))
    return out


# speedup vs baseline: 2.8390x; 1.0167x over previous
import functools

import jax
import jax.numpy as jnp
from jax import lax
from jax.experimental import pallas as pl
from jax.experimental.pallas import tpu as pltpu
from jax.experimental.pallas import tpu_sc as plsc

N = 10000
E = 320000
D = 128
C = 10

NC = 2
NS = 16
NW = NC * NS
EPW = E // NW

K = 40
NCHUNK = EPW // K
NSLOT = 5
NITER = NCHUNK // NSLOT

DK = 80
DCHUNK = EPW // DK
DGRP = 5

NP = 10240
RPT = NP // NS
RCH = K
DPT = NP // NS

_mesh = plsc.VectorSubcoreMesh(
    core_axis_name="c", subcore_axis_name="s", num_cores=NC, num_subcores=NS)


@functools.partial(
    pl.kernel,
    mesh=_mesh,
    out_type=jax.ShapeDtypeStruct((NC, 2, NP), jnp.float32),
    scratch_types=[
        pltpu.VMEM((DCHUNK, DK), jnp.int32),
        pltpu.VMEM((DCHUNK, DK), jnp.int32),
        pltpu.VMEM((DK,), jnp.float32),
        pltpu.VMEM((DPT,), jnp.float32),
        pltpu.VMEM_SHARED((NP,), jnp.float32),
        pltpu.VMEM_SHARED((NP,), jnp.float32),
        pltpu.SemaphoreType.DMA,
        pltpu.SemaphoreType.DMA,
    ],
)
def _degrees(src_hbm, dst_hbm, out_hbm, sidx, didx, ones_v, buf, odeg, ideg,
             sem_o, sem_i):
    cid = lax.axis_index("c")
    sid = lax.axis_index("s")
    wid = cid * NS + sid

    def zb(i, _):
        buf[pl.ds(i * 16, 16)] = jnp.zeros((16,), jnp.float32)
        return 0
    lax.fori_loop(0, DPT // 16, zb, 0)

    def ob(i, _):
        ones_v[pl.ds(i * 16, 16)] = jnp.ones((16,), jnp.float32)
        return 0
    lax.fori_loop(0, DK // 16, ob, 0)

    base = sid * DPT
    pltpu.sync_copy(buf, odeg.at[pl.ds(base, DPT)])
    pltpu.sync_copy(buf, ideg.at[pl.ds(base, DPT)])
    plsc.subcore_barrier()

    pltpu.sync_copy(src_hbm.at[wid], sidx)
    pltpu.sync_copy(dst_hbm.at[wid], didx)

    def step(g, _):
        for t in range(DGRP):
            c = g * DGRP + t
            pltpu.async_copy(ones_v, odeg.at[sidx.at[c]], sem_o, add=True)
            pltpu.async_copy(ones_v, ideg.at[didx.at[c]], sem_i, add=True)
        for t in range(DGRP):
            c = g * DGRP + t
            pltpu.make_async_copy(ones_v, odeg.at[sidx.at[c]], sem_o).wait()
            pltpu.make_async_copy(ones_v, ideg.at[didx.at[c]], sem_i).wait()
        return 0
    lax.fori_loop(0, DCHUNK // DGRP, step, 0)
    plsc.subcore_barrier()

    pltpu.sync_copy(odeg.at[pl.ds(base, DPT)], buf)
    pltpu.sync_copy(buf, out_hbm.at[cid, 0, pl.ds(base, DPT)])
    pltpu.sync_copy(ideg.at[pl.ds(base, DPT)], buf)
    pltpu.sync_copy(buf, out_hbm.at[cid, 1, pl.ds(base, DPT)])


@functools.partial(
    pl.kernel,
    mesh=_mesh,
    out_type=jax.ShapeDtypeStruct((NC, NP, D), jnp.float32),
    scratch_types=[
        [pltpu.VMEM((K,), jnp.int32) for _ in range(NSLOT)],
        [pltpu.VMEM((K,), jnp.int32) for _ in range(NSLOT)],
        [pltpu.VMEM((K, D), jnp.float32) for _ in range(NSLOT)],
        [pltpu.SemaphoreType.DMA for _ in range(NSLOT)],
        [pltpu.SemaphoreType.DMA for _ in range(NSLOT)],
        [pltpu.SemaphoreType.DMA for _ in range(NSLOT)],
        pltpu.VMEM_SHARED((NP, D), jnp.float32),
    ],
)
def _propagate(h_hbm, src_hbm, dst_hbm, out_hbm, sidx, didx, rows, isem, gsem,
               ssem, acc):
    cid = lax.axis_index("c")
    sid = lax.axis_index("s")
    wid = cid * NS + sid

    def zb(i, _):
        rows[0][i // 8, pl.ds((i % 8) * 16, 16)] = jnp.zeros((16,), jnp.float32)
        return 0
    lax.fori_loop(0, RCH * (D // 16), zb, 0)

    base = sid * RPT
    for r in range(RPT // RCH):
        pltpu.sync_copy(rows[0], acc.at[pl.ds(base + r * RCH, RCH)])
    plsc.subcore_barrier()

    for s in range(NSLOT):
        pltpu.async_copy(src_hbm.at[wid, s], sidx[s], isem[s])
        pltpu.async_copy(dst_hbm.at[wid, s], didx[s], isem[s])
    for s in range(NSLOT):
        pltpu.make_async_copy(src_hbm.at[wid, s], sidx[s], isem[s]).wait()
        pltpu.make_async_copy(dst_hbm.at[wid, s], didx[s], isem[s]).wait()
        pltpu.async_copy(h_hbm.at[sidx[s]], rows[s], gsem[s])

    def body(j, _):
        for s in range(NSLOT):
            pltpu.make_async_copy(h_hbm.at[sidx[s]], rows[s], gsem[s]).wait()
            pltpu.async_copy(rows[s], acc.at[didx[s]], ssem[s], add=True)

        @pl.when(j < NITER - 1)
        def _():
            c0 = (j + 1) * NSLOT
            for s in range(NSLOT):
                pltpu.make_async_copy(rows[s], acc.at[didx[s]], ssem[s]).wait()
                pltpu.async_copy(src_hbm.at[wid, c0 + s], sidx[s], isem[s])
                pltpu.async_copy(dst_hbm.at[wid, c0 + s], didx[s], isem[s])
            for s in range(NSLOT):
                pltpu.make_async_copy(src_hbm.at[wid, c0 + s], sidx[s], isem[s]).wait()
                pltpu.make_async_copy(dst_hbm.at[wid, c0 + s], didx[s], isem[s]).wait()
                pltpu.async_copy(h_hbm.at[sidx[s]], rows[s], gsem[s])
        return 0
    lax.fori_loop(0, NITER, body, 0)
    for s in range(NSLOT):
        pltpu.make_async_copy(rows[s], acc.at[didx[s]], ssem[s]).wait()
    plsc.subcore_barrier()

    for r in range(RPT // RCH):
        sl = pl.ds(base + r * RCH, RCH)
        pltpu.sync_copy(acc.at[sl], rows[0])
        pltpu.sync_copy(rows[0], out_hbm.at[cid, sl])


def _prep_body(degT_ref, x_ref, h0_ref, ns_ref, nd_ref):
    d = degT_ref[...]
    od = d[:, 0:1] + d[:, 1:2]
    idg = d[:, 2:3] + d[:, 3:4]
    ns = lax.rsqrt(jnp.maximum(od, 1.0))
    nd = lax.rsqrt(jnp.maximum(idg, 1.0))
    h0_ref[...] = x_ref[...] * ns
    ns_ref[...] = ns
    nd_ref[...] = nd


_prep = pl.pallas_call(
    _prep_body,
    out_shape=[
        jax.ShapeDtypeStruct((N, D), jnp.float32),
        jax.ShapeDtypeStruct((N, 1), jnp.float32),
        jax.ShapeDtypeStruct((N, 1), jnp.float32),
    ],
)


def _mid_body(p_ref, ns_ref, nd_ref, w_ref, b_ref, out_ref):
    agg = (p_ref[0, :N] + p_ref[1, :N]) * nd_ref[...]
    z = jnp.dot(agg, w_ref[...], preferred_element_type=jnp.float32) + b_ref[...]
    out_ref[...] = jnp.maximum(z, 0.0) * ns_ref[...]


_mid = pl.pallas_call(
    _mid_body,
    out_shape=jax.ShapeDtypeStruct((N, D), jnp.float32),
)


def _final_body(p_ref, nd_ref, w_ref, b_ref, wl_ref, bl_ref, out_ref):
    agg = (p_ref[0, :N] + p_ref[1, :N]) * nd_ref[...]
    z = jnp.dot(agg, w_ref[...], preferred_element_type=jnp.float32) + b_ref[...]
    h = jnp.maximum(z, 0.0)
    m = jnp.sum(h, axis=0, keepdims=True) * (1.0 / N)
    lg = jnp.dot(m, wl_ref[...], preferred_element_type=jnp.float32) + bl_ref[...]
    e = jnp.exp(lg - jnp.max(lg, axis=1, keepdims=True))
    out_ref[...] = e / jnp.sum(e, axis=1, keepdims=True)


_final = pl.pallas_call(
    _final_body,
    out_shape=jax.ShapeDtypeStruct((1, C), jnp.float32),
)


def kernel(x, edge_index, W1, b1, W2, b2, Wl, bl):
    src_p = edge_index[0].reshape(NW, NCHUNK, K)
    dst_p = edge_index[1].reshape(NW, NCHUNK, K)
    src_d = edge_index[0].reshape(NW, DCHUNK, DK)
    dst_d = edge_index[1].reshape(NW, DCHUNK, DK)

    deg_parts = _degrees(src_d, dst_d)
    degT = deg_parts[:, :, :N].transpose(2, 1, 0).reshape(N, 4)

    h0, ns, nd = _prep(degT, x)
    parts1 = _propagate(h0, src_p, dst_p)
    h1 = _mid(parts1, ns, nd, W1, b1.reshape(1, D))
    parts2 = _propagate(h1, src_p, dst_p)
    out = _final(parts2, nd, W2, b2.reshape(1, D), Wl, bl.reshape(1, C))
    return out


# speedup vs baseline: 2.8630x; 1.0085x over previous
import functools

import jax
import jax.numpy as jnp
from jax import lax
from jax.experimental import pallas as pl
from jax.experimental.pallas import tpu as pltpu
from jax.experimental.pallas import tpu_sc as plsc

N = 10000
E = 320000
D = 128
C = 10

NC = 2
NS = 16
NW = NC * NS
EPW = E // NW

K = 80
NCHUNK = EPW // K
NSLOT = 3
NITER = NCHUNK // NSLOT
TAIL = NCHUNK - NITER * NSLOT

DK = 80
DCHUNK = EPW // DK
DGRP = 5

NP = 10240
RPT = NP // NS
RCH = K
DPT = NP // NS

_mesh = plsc.VectorSubcoreMesh(
    core_axis_name="c", subcore_axis_name="s", num_cores=NC, num_subcores=NS)


@functools.partial(
    pl.kernel,
    mesh=_mesh,
    out_type=jax.ShapeDtypeStruct((NC, 2, NP), jnp.float32),
    scratch_types=[
        pltpu.VMEM((DCHUNK, DK), jnp.int32),
        pltpu.VMEM((DCHUNK, DK), jnp.int32),
        pltpu.VMEM((DK,), jnp.float32),
        pltpu.VMEM((DPT,), jnp.float32),
        pltpu.VMEM_SHARED((NP,), jnp.float32),
        pltpu.VMEM_SHARED((NP,), jnp.float32),
        pltpu.SemaphoreType.DMA,
        pltpu.SemaphoreType.DMA,
    ],
)
def _degrees(src_hbm, dst_hbm, out_hbm, sidx, didx, ones_v, buf, odeg, ideg,
             sem_o, sem_i):
    cid = lax.axis_index("c")
    sid = lax.axis_index("s")
    wid = cid * NS + sid

    def zb(i, _):
        buf[pl.ds(i * 16, 16)] = jnp.zeros((16,), jnp.float32)
        return 0
    lax.fori_loop(0, DPT // 16, zb, 0)

    def ob(i, _):
        ones_v[pl.ds(i * 16, 16)] = jnp.ones((16,), jnp.float32)
        return 0
    lax.fori_loop(0, DK // 16, ob, 0)

    base = sid * DPT
    pltpu.sync_copy(buf, odeg.at[pl.ds(base, DPT)])
    pltpu.sync_copy(buf, ideg.at[pl.ds(base, DPT)])
    plsc.subcore_barrier()

    pltpu.sync_copy(src_hbm.at[wid], sidx)
    pltpu.sync_copy(dst_hbm.at[wid], didx)

    def step(g, _):
        for t in range(DGRP):
            c = g * DGRP + t
            pltpu.async_copy(ones_v, odeg.at[sidx.at[c]], sem_o, add=True)
            pltpu.async_copy(ones_v, ideg.at[didx.at[c]], sem_i, add=True)
        for t in range(DGRP):
            c = g * DGRP + t
            pltpu.make_async_copy(ones_v, odeg.at[sidx.at[c]], sem_o).wait()
            pltpu.make_async_copy(ones_v, ideg.at[didx.at[c]], sem_i).wait()
        return 0
    lax.fori_loop(0, DCHUNK // DGRP, step, 0)
    plsc.subcore_barrier()

    pltpu.sync_copy(odeg.at[pl.ds(base, DPT)], buf)
    pltpu.sync_copy(buf, out_hbm.at[cid, 0, pl.ds(base, DPT)])
    pltpu.sync_copy(ideg.at[pl.ds(base, DPT)], buf)
    pltpu.sync_copy(buf, out_hbm.at[cid, 1, pl.ds(base, DPT)])


@functools.partial(
    pl.kernel,
    mesh=_mesh,
    out_type=jax.ShapeDtypeStruct((NC, NP, D), jnp.float32),
    scratch_types=[
        [pltpu.VMEM((1, K), jnp.int32) for _ in range(NSLOT)],
        [pltpu.VMEM((1, K), jnp.int32) for _ in range(NSLOT)],
        [pltpu.VMEM((K, D), jnp.float32) for _ in range(NSLOT)],
        [pltpu.SemaphoreType.DMA for _ in range(NSLOT)],
        [pltpu.SemaphoreType.DMA for _ in range(NSLOT)],
        [pltpu.SemaphoreType.DMA for _ in range(NSLOT)],
        pltpu.VMEM_SHARED((NP, D), jnp.float32),
    ],
)
def _propagate(h_hbm, src_hbm, dst_hbm, out_hbm, sidx, didx, rows, isem, gsem,
               ssem, acc):
    cid = lax.axis_index("c")
    sid = lax.axis_index("s")
    wid = cid * NS + sid

    def zb(i, _):
        rows[0][i // 8, pl.ds((i % 8) * 16, 16)] = jnp.zeros((16,), jnp.float32)
        return 0
    lax.fori_loop(0, RCH * (D // 16), zb, 0)

    base = sid * RPT
    for r in range(RPT // RCH):
        pltpu.async_copy(rows[0], acc.at[pl.ds(base + r * RCH, RCH)], isem[0])
    for r in range(RPT // RCH):
        pltpu.make_async_copy(rows[0], acc.at[pl.ds(base, RCH)], isem[0]).wait()
    plsc.subcore_barrier()

    for s in range(NSLOT):
        pltpu.async_copy(src_hbm.at[wid, pl.ds(s, 1)], sidx[s], isem[s])
        pltpu.async_copy(dst_hbm.at[wid, pl.ds(s, 1)], didx[s], isem[s])
    for s in range(NSLOT):
        pltpu.make_async_copy(src_hbm.at[wid, pl.ds(s, 1)], sidx[s], isem[s]).wait()
        pltpu.make_async_copy(dst_hbm.at[wid, pl.ds(s, 1)], didx[s], isem[s]).wait()
        pltpu.async_copy(h_hbm.at[sidx[s].at[0]], rows[s], gsem[s])

    def body(j, _):
        for s in range(NSLOT):
            pltpu.make_async_copy(h_hbm.at[sidx[s].at[0]], rows[s], gsem[s]).wait()
            pltpu.async_copy(rows[s], acc.at[didx[s].at[0]], ssem[s], add=True)

        @pl.when(j < NITER - 1)
        def _():
            c0 = (j + 1) * NSLOT
            for s in range(NSLOT):
                pltpu.make_async_copy(rows[s], acc.at[didx[s].at[0]], ssem[s]).wait()
                pltpu.async_copy(src_hbm.at[wid, pl.ds(c0 + s, 1)], sidx[s], isem[s])
                pltpu.async_copy(dst_hbm.at[wid, pl.ds(c0 + s, 1)], didx[s], isem[s])
            for s in range(NSLOT):
                pltpu.make_async_copy(src_hbm.at[wid, pl.ds(c0 + s, 1)], sidx[s], isem[s]).wait()
                pltpu.make_async_copy(dst_hbm.at[wid, pl.ds(c0 + s, 1)], didx[s], isem[s]).wait()
                pltpu.async_copy(h_hbm.at[sidx[s].at[0]], rows[s], gsem[s])
        return 0
    lax.fori_loop(0, NITER, body, 0)

    for s in range(TAIL):
        c = NITER * NSLOT + s
        pltpu.make_async_copy(rows[s], acc.at[didx[s].at[0]], ssem[s]).wait()
        pltpu.async_copy(src_hbm.at[wid, pl.ds(c, 1)], sidx[s], isem[s])
        pltpu.async_copy(dst_hbm.at[wid, pl.ds(c, 1)], didx[s], isem[s])
    for s in range(TAIL):
        c = NITER * NSLOT + s
        pltpu.make_async_copy(src_hbm.at[wid, pl.ds(c, 1)], sidx[s], isem[s]).wait()
        pltpu.make_async_copy(dst_hbm.at[wid, pl.ds(c, 1)], didx[s], isem[s]).wait()
        pltpu.async_copy(h_hbm.at[sidx[s].at[0]], rows[s], gsem[s])
    for s in range(TAIL):
        pltpu.make_async_copy(h_hbm.at[sidx[s].at[0]], rows[s], gsem[s]).wait()
        pltpu.async_copy(rows[s], acc.at[didx[s].at[0]], ssem[s], add=True)
    for s in range(NSLOT):
        pltpu.make_async_copy(rows[s], acc.at[didx[s].at[0]], ssem[s]).wait()
    plsc.subcore_barrier()

    def osl(r):
        return pl.ds(base + r * RCH, RCH)

    for r in range(RPT // RCH):
        b = rows[r % 2]
        if r >= 2:
            pltpu.make_async_copy(b, out_hbm.at[cid, osl(r - 2)], gsem[r % 2]).wait()
        pltpu.sync_copy(acc.at[osl(r)], b)
        pltpu.async_copy(b, out_hbm.at[cid, osl(r)], gsem[r % 2])
    for r in range(RPT // RCH - 2, RPT // RCH):
        pltpu.make_async_copy(rows[r % 2], out_hbm.at[cid, osl(r)], gsem[r % 2]).wait()


def _prep_body(degT_ref, x_ref, h0_ref, ns_ref, nd_ref):
    d = degT_ref[...]
    od = d[:, 0:1] + d[:, 1:2]
    idg = d[:, 2:3] + d[:, 3:4]
    ns = lax.rsqrt(jnp.maximum(od, 1.0))
    nd = lax.rsqrt(jnp.maximum(idg, 1.0))
    h0_ref[...] = x_ref[...] * ns
    ns_ref[...] = ns
    nd_ref[...] = nd


_prep = pl.pallas_call(
    _prep_body,
    out_shape=[
        jax.ShapeDtypeStruct((N, D), jnp.float32),
        jax.ShapeDtypeStruct((N, 1), jnp.float32),
        jax.ShapeDtypeStruct((N, 1), jnp.float32),
    ],
)


def _mid_body(p_ref, ns_ref, nd_ref, w_ref, b_ref, out_ref):
    agg = (p_ref[0, :N] + p_ref[1, :N]) * nd_ref[...]
    z = jnp.dot(agg, w_ref[...], preferred_element_type=jnp.float32) + b_ref[...]
    out_ref[...] = jnp.maximum(z, 0.0) * ns_ref[...]


_mid = pl.pallas_call(
    _mid_body,
    out_shape=jax.ShapeDtypeStruct((N, D), jnp.float32),
)


def _final_body(p_ref, nd_ref, w_ref, b_ref, wl_ref, bl_ref, out_ref):
    agg = (p_ref[0, :N] + p_ref[1, :N]) * nd_ref[...]
    z = jnp.dot(agg, w_ref[...], preferred_element_type=jnp.float32) + b_ref[...]
    h = jnp.maximum(z, 0.0)
    m = jnp.sum(h, axis=0, keepdims=True) * (1.0 / N)
    lg = jnp.dot(m, wl_ref[...], preferred_element_type=jnp.float32) + bl_ref[...]
    e = jnp.exp(lg - jnp.max(lg, axis=1, keepdims=True))
    out_ref[...] = e / jnp.sum(e, axis=1, keepdims=True)


_final = pl.pallas_call(
    _final_body,
    out_shape=jax.ShapeDtypeStruct((1, C), jnp.float32),
)


def kernel(x, edge_index, W1, b1, W2, b2, Wl, bl):
    src_p = edge_index[0].reshape(NW, NCHUNK, K)
    dst_p = edge_index[1].reshape(NW, NCHUNK, K)
    src_d = edge_index[0].reshape(NW, DCHUNK, DK)
    dst_d = edge_index[1].reshape(NW, DCHUNK, DK)

    deg_parts = _degrees(src_d, dst_d)
    degT = deg_parts[:, :, :N].transpose(2, 1, 0).reshape(N, 4)

    h0, ns, nd = _prep(degT, x)
    parts1 = _propagate(h0, src_p, dst_p)
    h1 = _mid(parts1, ns, nd, W1, b1.reshape(1, D))
    parts2 = _propagate(h1, src_p, dst_p)
    out = _final(parts2, nd, W2, b2.reshape(1, D), Wl, bl.reshape(1, C))
    return out
